# Initial kernel scaffold; baseline (speedup 1.0000x reference)
import jax
import jax.numpy as jnp
from jax.experimental import pallas as pl


def kernel(x, edge_index, edge_attr, batch, We0, be0, W0, b0, We1, be1, W1, b1, We2, be2, W2, b2, Wh, bh, Wv, bv):
    raise NotImplementedError("write your pallas kernel here")


# speedup vs baseline: 2.0859x; 2.0859x over previous
import functools

import jax
import jax.numpy as jnp
from jax import lax
from jax.experimental import pallas as pl
from jax.experimental.pallas import tpu as pltpu
from jax.experimental.pallas import tpu_sc as plsc

N_NODES = 10000
N_EDGES = 160000
D = 256
DH = 128
NG = 64

EB = 2000
NB = 2000
SCB = 80
N_SUB = 16
EPT = N_EDGES // N_SUB
NWR = 10
RPT = N_NODES // NWR
ZROWS = 200

_f32 = jnp.float32


def _edge_body(ea_ref, we_ref, be_ref, out_ref):
    acc = jnp.dot(ea_ref[...], we_ref[...], preferred_element_type=_f32)
    out_ref[...] = jnp.maximum(acc + be_ref[...], 0.0)


def _edge_mlp(edge_attr, We, be2):
    nblk = N_EDGES // EB
    return pl.pallas_call(
        _edge_body,
        grid=(2, nblk),
        in_specs=[
            pl.BlockSpec((EB, 16), lambda c, j: (j, 0)),
            pl.BlockSpec((16, DH), lambda c, j: (0, c)),
            pl.BlockSpec((1, DH), lambda c, j: (0, c)),
        ],
        out_specs=pl.BlockSpec((EB, DH), lambda c, j, _n=nblk: (c * _n + j, 0)),
        out_shape=jax.ShapeDtypeStruct((2 * N_EDGES, DH), _f32),
    )(edge_attr, We, be2)


_sc_mesh = plsc.VectorSubcoreMesh(core_axis_name="c", subcore_axis_name="s")


@functools.partial(
    pl.kernel,
    out_type=jax.ShapeDtypeStruct((2 * N_NODES, DH), _f32),
    mesh=_sc_mesh,
    scratch_types=[
        pltpu.VMEM((SCB,), jnp.int32),
        pltpu.VMEM((SCB,), jnp.int32),
        pltpu.VMEM((SCB, DH), _f32),
        pltpu.VMEM((SCB, DH), _f32),
        pltpu.VMEM((ZROWS, DH), _f32),
        pltpu.VMEM_SHARED((N_NODES, DH), _f32),
        pltpu.SemaphoreType.DMA,
    ],
)
def _sc_edge(h_hbm, e_hbm, src_hbm, dst_hbm, out_hbm,
             idx_s, idx_d, gath_v, e_v, zero_v, shared, sem):
    c = lax.axis_index("c")
    s = lax.axis_index("s")

    def zrow(r, carry):
        for t in range(DH // 16):
            zero_v[r, pl.ds(t * 16, 16)] = jnp.zeros((16,), _f32)
        return carry

    @pl.when(s < NWR)
    def _zero():
        lax.fori_loop(0, ZROWS, zrow, 0)
        for i in range(RPT // ZROWS):
            pltpu.sync_copy(zero_v, shared.at[pl.ds(s * RPT + i * ZROWS, ZROWS)])

    plsc.subcore_barrier()

    ebase = s * EPT

    def blk(j, carry):
        off = ebase + j * SCB
        pltpu.sync_copy(src_hbm.at[pl.ds(off, SCB)], idx_s)
        pltpu.sync_copy(dst_hbm.at[pl.ds(off, SCB)], idx_d)
        for t in range(SCB // 16):
            idx_s[pl.ds(t * 16, 16)] = idx_s[pl.ds(t * 16, 16)] + c * N_NODES
        pltpu.async_copy(h_hbm.at[idx_s], gath_v, sem).wait()
        pltpu.sync_copy(e_hbm.at[pl.ds(c * N_EDGES + off, SCB)], e_v)

        def row(r, rc):
            for t in range(DH // 16):
                g = gath_v[r, pl.ds(t * 16, 16)]
                ee = e_v[r, pl.ds(t * 16, 16)]
                gath_v[r, pl.ds(t * 16, 16)] = jnp.maximum(g + ee, 0.0)
            return rc

        lax.fori_loop(0, SCB, row, 0)
        pltpu.sync_copy(gath_v, shared.at[idx_d], add=True)
        return carry

    lax.fori_loop(0, EPT // SCB, blk, 0)
    plsc.subcore_barrier()

    @pl.when(s < NWR)
    def _writeout():
        for i in range(RPT // ZROWS):
            rb = s * RPT + i * ZROWS
            pltpu.sync_copy(shared.at[pl.ds(rb, ZROWS)],
                            out_hbm.at[pl.ds(c * N_NODES + rb, ZROWS)])


def _node_body(h_ref, agg_ref, w_ref, b_ref, out_ref):
    hp0 = h_ref[0] + agg_ref[0]
    hp1 = h_ref[1] + agg_ref[1]
    acc = jnp.dot(hp0, w_ref[0:DH, :], preferred_element_type=_f32)
    acc += jnp.dot(hp1, w_ref[DH:2 * DH, :], preferred_element_type=_f32)
    out_ref[...] = jnp.maximum(acc + b_ref[...], 0.0)


def _node_update(h3, agg3, W, b2):
    nblk = N_NODES // NB
    return pl.pallas_call(
        _node_body,
        grid=(2, nblk),
        in_specs=[
            pl.BlockSpec((2, NB, DH), lambda c2, j: (0, j, 0)),
            pl.BlockSpec((2, NB, DH), lambda c2, j: (0, j, 0)),
            pl.BlockSpec((D, DH), lambda c2, j: (0, c2)),
            pl.BlockSpec((1, DH), lambda c2, j: (0, c2)),
        ],
        out_specs=pl.BlockSpec((NB, DH), lambda c2, j, _n=nblk: (c2 * _n + j, 0)),
        out_shape=jax.ShapeDtypeStruct((2 * N_NODES, DH), _f32),
    )(h3, agg3, W, b2)


def _head_body(h_ref, bat_ref, wh_ref, bh_ref, wv_ref, bv_ref,
               log_ref, val_ref, s0, s1, cnt):
    j = pl.program_id(0)
    nb = pl.num_programs(0)
    h0 = h_ref[0]
    h1 = h_ref[1]
    lg = jnp.dot(h0, wh_ref[0:DH, :], preferred_element_type=_f32)
    lg += jnp.dot(h1, wh_ref[DH:2 * DH, :], preferred_element_type=_f32)
    log_ref[...] = lg + bh_ref[...]

    onehot = (bat_ref[...] == lax.broadcasted_iota(jnp.int32, (1, NG), 1))
    onehot = onehot.astype(_f32)
    dn = (((0,), (0,)), ((), ()))
    ps0 = lax.dot_general(onehot, h0, dn, preferred_element_type=_f32)
    ps1 = lax.dot_general(onehot, h1, dn, preferred_element_type=_f32)
    pc = lax.dot_general(onehot, jnp.ones((NB, 1), _f32), dn,
                         preferred_element_type=_f32)

    @pl.when(j == 0)
    def _init():
        s0[...] = ps0
        s1[...] = ps1
        cnt[...] = pc

    @pl.when(j > 0)
    def _acc():
        s0[...] += ps0
        s1[...] += ps1
        cnt[...] += pc

    @pl.when(j == nb - 1)
    def _fin():
        v = jnp.dot(s0[...], wv_ref[0:DH, :], preferred_element_type=_f32)
        v += jnp.dot(s1[...], wv_ref[DH:2 * DH, :], preferred_element_type=_f32)
        v = v / jnp.maximum(cnt[...], 1.0) + bv_ref[...]
        val_ref[...] = jnp.tanh(v)


def _head(h3, batch2, Wh, bh2, Wv, bv2):
    return pl.pallas_call(
        _head_body,
        grid=(N_NODES // NB,),
        in_specs=[
            pl.BlockSpec((2, NB, DH), lambda j: (0, j, 0)),
            pl.BlockSpec((NB, 1), lambda j: (j, 0)),
            pl.BlockSpec((D, 1), lambda j: (0, 0)),
            pl.BlockSpec((1, 1), lambda j: (0, 0)),
            pl.BlockSpec((D, 1), lambda j: (0, 0)),
            pl.BlockSpec((1, 1), lambda j: (0, 0)),
        ],
        out_specs=[
            pl.BlockSpec((NB, 1), lambda j: (j, 0)),
            pl.BlockSpec((NG, 1), lambda j: (0, 0)),
        ],
        out_shape=[
            jax.ShapeDtypeStruct((N_NODES, 1), _f32),
            jax.ShapeDtypeStruct((NG, 1), _f32),
        ],
        scratch_shapes=[
            pltpu.VMEM((NG, DH), _f32),
            pltpu.VMEM((NG, DH), _f32),
            pltpu.VMEM((NG, 1), _f32),
        ],
    )(h3, batch2, Wh, bh2, Wv, bv2)


def kernel(x, edge_index, edge_attr, batch,
           We0, be0, W0, b0, We1, be1, W1, b1, We2, be2, W2, b2,
           Wh, bh, Wv, bv):
    src = edge_index[0].astype(jnp.int32)
    dst = edge_index[1].astype(jnp.int32)
    batch2 = batch.astype(jnp.int32).reshape(N_NODES, 1)

    h = jnp.concatenate([x[:, :DH], x[:, DH:]], axis=0)
    layers = [(We0, be0, W0, b0), (We1, be1, W1, b1), (We2, be2, W2, b2)]
    for We, be, W, b in layers:
        e_flat = _edge_mlp(edge_attr, We, be.reshape(1, D))
        agg = _sc_edge(h, e_flat, src, dst)
        h = _node_update(h.reshape(2, N_NODES, DH),
                         agg.reshape(2, N_NODES, DH), W, b.reshape(1, D))

    logits2, value2 = _head(h.reshape(2, N_NODES, DH), batch2,
                            Wh, bh.reshape(1, 1), Wv, bv.reshape(1, 1))
    return logits2.ravel(), value2.ravel()


# speedup vs baseline: 3.0328x; 1.4540x over previous
import functools

import jax
import jax.numpy as jnp
from jax import lax
from jax.experimental import pallas as pl
from jax.experimental.pallas import tpu as pltpu
from jax.experimental.pallas import tpu_sc as plsc

N_NODES = 10000
N_EDGES = 160000
D = 256
DH = 128
NG = 64

EB = 2000
NB = 2000
SCB = 80
N_SUB = 16
EPT = N_EDGES // N_SUB
NWR = 10
RPT = N_NODES // NWR
ZROWS = 40
WROWS = 200

_f32 = jnp.float32


def _edge_body(ea_ref, we_ref, be_ref, out_ref):
    acc = jnp.dot(ea_ref[...], we_ref[...], preferred_element_type=_f32)
    out_ref[...] = jnp.maximum(acc + be_ref[...], 0.0)


def _edge_mlp(edge_attr, We, be2):
    nblk = N_EDGES // EB
    return pl.pallas_call(
        _edge_body,
        grid=(2, nblk),
        in_specs=[
            pl.BlockSpec((EB, 16), lambda c, j: (j, 0)),
            pl.BlockSpec((16, DH), lambda c, j: (0, c)),
            pl.BlockSpec((1, DH), lambda c, j: (0, c)),
        ],
        out_specs=pl.BlockSpec((EB, DH), lambda c, j, _n=nblk: (c * _n + j, 0)),
        out_shape=jax.ShapeDtypeStruct((2 * N_EDGES, DH), _f32),
    )(edge_attr, We, be2)


_sc_mesh = plsc.VectorSubcoreMesh(core_axis_name="c", subcore_axis_name="s")


@functools.partial(
    pl.kernel,
    out_type=jax.ShapeDtypeStruct((2 * N_NODES, DH), _f32),
    mesh=_sc_mesh,
    scratch_types=[
        pltpu.VMEM((SCB,), jnp.int32),
        pltpu.VMEM((SCB,), jnp.int32),
        pltpu.VMEM((SCB,), jnp.int32),
        pltpu.VMEM((SCB,), jnp.int32),
        pltpu.VMEM((SCB, DH), _f32),
        pltpu.VMEM((SCB, DH), _f32),
        pltpu.VMEM((SCB, DH), _f32),
        pltpu.VMEM((SCB, DH), _f32),
        pltpu.VMEM((ZROWS, DH), _f32),
        pltpu.VMEM_SHARED((N_NODES, DH), _f32),
        pltpu.SemaphoreType.DMA,
        pltpu.SemaphoreType.DMA,
    ],
)
def _sc_edge(h_hbm, e_hbm, src_hbm, dst_hbm, out_hbm,
             idx_s0, idx_d0, idx_s1, idx_d1, g0, g1, e0, e1,
             zero_v, shared, sem0, sem1):
    c = lax.axis_index("c")
    s = lax.axis_index("s")
    idx_s = (idx_s0, idx_s1)
    idx_d = (idx_d0, idx_d1)
    gath = (g0, g1)
    e_v = (e0, e1)
    sem = (sem0, sem1)

    def zrow(r, carry):
        for t in range(DH // 16):
            zero_v[r, pl.ds(t * 16, 16)] = jnp.zeros((16,), _f32)
        return carry

    @pl.when(s < NWR)
    def _zero():
        lax.fori_loop(0, ZROWS, zrow, 0)
        for i in range(RPT // ZROWS):
            pltpu.sync_copy(zero_v, shared.at[pl.ds(s * RPT + i * ZROWS, ZROWS)])

    plsc.subcore_barrier()

    ebase = s * EPT

    def start(j, b):
        off = ebase + j * SCB
        pltpu.sync_copy(src_hbm.at[pl.ds(off, SCB)], idx_s[b])
        pltpu.sync_copy(dst_hbm.at[pl.ds(off, SCB)], idx_d[b])
        for t in range(SCB // 16):
            sl = pl.ds(t * 16, 16)
            idx_s[b][sl] = idx_s[b][sl] + c * N_NODES
        pltpu.async_copy(h_hbm.at[idx_s[b]], gath[b], sem[b])
        pltpu.async_copy(e_hbm.at[pl.ds(c * N_EDGES + off, SCB)], e_v[b], sem[b])

    def finish(j, b):
        off = ebase + j * SCB
        pltpu.make_async_copy(h_hbm.at[idx_s[b]], gath[b], sem[b]).wait()
        pltpu.make_async_copy(
            e_hbm.at[pl.ds(c * N_EDGES + off, SCB)], e_v[b], sem[b]).wait()

        @plsc.parallel_loop(0, SCB, unroll=4)
        def _row(r):
            for t in range(DH // 16):
                sl = pl.ds(t * 16, 16)
                gath[b][r, sl] = jnp.maximum(gath[b][r, sl] + e_v[b][r, sl], 0.0)

        pltpu.sync_copy(gath[b], shared.at[idx_d[b]], add=True)

    nblk = EPT // SCB
    start(0, 0)

    def pipe(t, carry):
        start(2 * t + 1, 1)
        finish(2 * t, 0)
        start(2 * t + 2, 0)
        finish(2 * t + 1, 1)
        return carry

    lax.fori_loop(0, (nblk - 1) // 2, pipe, 0)
    finish(nblk - 1, 0)
    plsc.subcore_barrier()

    @pl.when(s < NWR)
    def _writeout():
        for i in range(RPT // WROWS):
            rb = s * RPT + i * WROWS
            pltpu.sync_copy(shared.at[pl.ds(rb, WROWS)],
                            out_hbm.at[pl.ds(c * N_NODES + rb, WROWS)])


def _node_body(h_ref, agg_ref, w_ref, b_ref, out_ref):
    hp0 = h_ref[0] + agg_ref[0]
    hp1 = h_ref[1] + agg_ref[1]
    acc = jnp.dot(hp0, w_ref[0:DH, :], preferred_element_type=_f32)
    acc += jnp.dot(hp1, w_ref[DH:2 * DH, :], preferred_element_type=_f32)
    out_ref[...] = jnp.maximum(acc + b_ref[...], 0.0)


def _node_update(h3, agg3, W, b2):
    nblk = N_NODES // NB
    return pl.pallas_call(
        _node_body,
        grid=(2, nblk),
        in_specs=[
            pl.BlockSpec((2, NB, DH), lambda c2, j: (0, j, 0)),
            pl.BlockSpec((2, NB, DH), lambda c2, j: (0, j, 0)),
            pl.BlockSpec((D, DH), lambda c2, j: (0, c2)),
            pl.BlockSpec((1, DH), lambda c2, j: (0, c2)),
        ],
        out_specs=pl.BlockSpec((NB, DH), lambda c2, j, _n=nblk: (c2 * _n + j, 0)),
        out_shape=jax.ShapeDtypeStruct((2 * N_NODES, DH), _f32),
    )(h3, agg3, W, b2)


def _head_body(h_ref, bat_ref, wh_ref, bh_ref, wv_ref, bv_ref,
               log_ref, val_ref, s0, s1, cnt):
    j = pl.program_id(0)
    nb = pl.num_programs(0)
    h0 = h_ref[0]
    h1 = h_ref[1]
    lg = jnp.dot(h0, wh_ref[0:DH, :], preferred_element_type=_f32)
    lg += jnp.dot(h1, wh_ref[DH:2 * DH, :], preferred_element_type=_f32)
    log_ref[...] = lg + bh_ref[...]

    onehot = (bat_ref[...] == lax.broadcasted_iota(jnp.int32, (1, NG), 1))
    onehot = onehot.astype(_f32)
    dn = (((0,), (0,)), ((), ()))
    ps0 = lax.dot_general(onehot, h0, dn, preferred_element_type=_f32)
    ps1 = lax.dot_general(onehot, h1, dn, preferred_element_type=_f32)
    pc = lax.dot_general(onehot, jnp.ones((NB, 1), _f32), dn,
                         preferred_element_type=_f32)

    @pl.when(j == 0)
    def _init():
        s0[...] = ps0
        s1[...] = ps1
        cnt[...] = pc

    @pl.when(j > 0)
    def _acc():
        s0[...] += ps0
        s1[...] += ps1
        cnt[...] += pc

    @pl.when(j == nb - 1)
    def _fin():
        v = jnp.dot(s0[...], wv_ref[0:DH, :], preferred_element_type=_f32)
        v += jnp.dot(s1[...], wv_ref[DH:2 * DH, :], preferred_element_type=_f32)
        v = v / jnp.maximum(cnt[...], 1.0) + bv_ref[...]
        val_ref[...] = jnp.tanh(v)


def _head(h3, batch2, Wh, bh2, Wv, bv2):
    return pl.pallas_call(
        _head_body,
        grid=(N_NODES // NB,),
        in_specs=[
            pl.BlockSpec((2, NB, DH), lambda j: (0, j, 0)),
            pl.BlockSpec((NB, 1), lambda j: (j, 0)),
            pl.BlockSpec((D, 1), lambda j: (0, 0)),
            pl.BlockSpec((1, 1), lambda j: (0, 0)),
            pl.BlockSpec((D, 1), lambda j: (0, 0)),
            pl.BlockSpec((1, 1), lambda j: (0, 0)),
        ],
        out_specs=[
            pl.BlockSpec((NB, 1), lambda j: (j, 0)),
            pl.BlockSpec((NG, 1), lambda j: (0, 0)),
        ],
        out_shape=[
            jax.ShapeDtypeStruct((N_NODES, 1), _f32),
            jax.ShapeDtypeStruct((NG, 1), _f32),
        ],
        scratch_shapes=[
            pltpu.VMEM((NG, DH), _f32),
            pltpu.VMEM((NG, DH), _f32),
            pltpu.VMEM((NG, 1), _f32),
        ],
    )(h3, batch2, Wh, bh2, Wv, bv2)


def kernel(x, edge_index, edge_attr, batch,
           We0, be0, W0, b0, We1, be1, W1, b1, We2, be2, W2, b2,
           Wh, bh, Wv, bv):
    src = edge_index[0].astype(jnp.int32)
    dst = edge_index[1].astype(jnp.int32)
    batch2 = batch.astype(jnp.int32).reshape(N_NODES, 1)

    h = jnp.concatenate([x[:, :DH], x[:, DH:]], axis=0)
    layers = [(We0, be0, W0, b0), (We1, be1, W1, b1), (We2, be2, W2, b2)]
    for We, be, W, b in layers:
        e_flat = _edge_mlp(edge_attr, We, be.reshape(1, D))
        agg = _sc_edge(h, e_flat, src, dst)
        h = _node_update(h.reshape(2, N_NODES, DH),
                         agg.reshape(2, N_NODES, DH), W, b.reshape(1, D))

    logits2, value2 = _head(h.reshape(2, N_NODES, DH), batch2,
                            Wh, bh.reshape(1, 1), Wv, bv.reshape(1, 1))
    return logits2.ravel(), value2.ravel()


# speedup vs baseline: 3.5210x; 1.1610x over previous
import functools

import jax
import jax.numpy as jnp
from jax import lax
from jax.experimental import pallas as pl
from jax.experimental.pallas import tpu as pltpu
from jax.experimental.pallas import tpu_sc as plsc

N_NODES = 10000
N_EDGES = 160000
D = 256
DH = 128
NG = 64

EB = 2000
NB = 2000
SCB = 80
N_SUB = 16
EPT = N_EDGES // N_SUB
NWR = 10
RPT = N_NODES // NWR
ZROWS = 40
WROWS = 200

_f32 = jnp.float32


def _edge_body(ea_ref, we_ref, be_ref, out_ref):
    acc = jnp.dot(ea_ref[...], we_ref[...], preferred_element_type=_f32)
    out_ref[...] = jnp.maximum(acc + be_ref[...], 0.0)


def _edge_mlp(edge_attr, We, be2):
    nblk = N_EDGES // EB
    return pl.pallas_call(
        _edge_body,
        grid=(2, nblk),
        in_specs=[
            pl.BlockSpec((EB, 16), lambda c, j: (j, 0)),
            pl.BlockSpec((16, DH), lambda c, j: (0, c)),
            pl.BlockSpec((1, DH), lambda c, j: (0, c)),
        ],
        out_specs=pl.BlockSpec((EB, DH), lambda c, j, _n=nblk: (c * _n + j, 0)),
        out_shape=jax.ShapeDtypeStruct((2 * N_EDGES, DH), _f32),
    )(edge_attr, We, be2)


_sc_mesh = plsc.VectorSubcoreMesh(core_axis_name="c", subcore_axis_name="s")


IB = 25
NBATCH = EPT // (IB * SCB)


@functools.partial(
    pl.kernel,
    out_type=jax.ShapeDtypeStruct((2 * N_NODES, DH), _f32),
    mesh=_sc_mesh,
    scratch_types=[
        pltpu.VMEM((IB * SCB,), jnp.int32),
        pltpu.VMEM((IB * SCB,), jnp.int32),
        pltpu.VMEM((SCB,), jnp.int32),
        pltpu.VMEM((SCB,), jnp.int32),
        pltpu.VMEM((SCB, DH), _f32),
        pltpu.VMEM((SCB, DH), _f32),
        pltpu.VMEM((SCB, DH), _f32),
        pltpu.VMEM((SCB, DH), _f32),
        pltpu.VMEM((ZROWS, DH), _f32),
        pltpu.VMEM_SHARED((N_NODES, DH), _f32),
        pltpu.SemaphoreType.DMA,
        pltpu.SemaphoreType.DMA,
    ],
)
def _sc_edge(h_hbm, e_hbm, src_hbm, dst_hbm, out_hbm,
             idx_sbig, idx_dbig, idx_d0, idx_d1, g0, g1, e0, e1,
             zero_v, shared, sem0, sem1):
    c = lax.axis_index("c")
    s = lax.axis_index("s")
    idx_d = (idx_d0, idx_d1)
    gath = (g0, g1)
    e_v = (e0, e1)
    sem = (sem0, sem1)

    def zrow(r, carry):
        for t in range(DH // 16):
            zero_v[r, pl.ds(t * 16, 16)] = jnp.zeros((16,), _f32)
        return carry

    @pl.when(s < NWR)
    def _zero():
        lax.fori_loop(0, ZROWS, zrow, 0)
        for i in range(RPT // ZROWS):
            pltpu.sync_copy(zero_v, shared.at[pl.ds(s * RPT + i * ZROWS, ZROWS)])

    plsc.subcore_barrier()

    ebase = s * EPT

    def batch(t, carry):
        boff = ebase + t * (IB * SCB)
        pltpu.sync_copy(src_hbm.at[pl.ds(boff, IB * SCB)], idx_sbig)
        pltpu.sync_copy(dst_hbm.at[pl.ds(boff, IB * SCB)], idx_dbig)

        def adj(i, ac):
            sl = pl.ds(i * 16, 16)
            idx_sbig[sl] = idx_sbig[sl] + c * N_NODES
            return ac

        lax.fori_loop(0, IB * SCB // 16, adj, 0)

        def start(k, b):
            for t in range(SCB // 16):
                idx_d[b][pl.ds(t * 16, 16)] = idx_dbig[pl.ds(k * SCB + t * 16, 16)]
            pltpu.async_copy(h_hbm.at[idx_sbig.at[pl.ds(k * SCB, SCB)]],
                             gath[b], sem[b])
            pltpu.async_copy(
                e_hbm.at[pl.ds(c * N_EDGES + boff + k * SCB, SCB)],
                e_v[b], sem[b])

        def finish(k, b):
            pltpu.make_async_copy(h_hbm.at[idx_sbig.at[pl.ds(k * SCB, SCB)]],
                                  gath[b], sem[b]).wait()
            pltpu.make_async_copy(
                e_hbm.at[pl.ds(c * N_EDGES + boff + k * SCB, SCB)],
                e_v[b], sem[b]).wait()

            @plsc.parallel_loop(0, SCB, unroll=4)
            def _row(r):
                for tt in range(DH // 16):
                    sl = pl.ds(tt * 16, 16)
                    gath[b][r, sl] = jnp.maximum(
                        gath[b][r, sl] + e_v[b][r, sl], 0.0)

            pltpu.sync_copy(gath[b], shared.at[idx_d[b]], add=True)

        start(0, 0)
        for k in range(IB - 1):
            start(k + 1, (k + 1) % 2)
            finish(k, k % 2)
        finish(IB - 1, (IB - 1) % 2)
        return carry

    lax.fori_loop(0, NBATCH, batch, 0)
    plsc.subcore_barrier()

    @pl.when(s < NWR)
    def _writeout():
        for i in range(RPT // WROWS):
            rb = s * RPT + i * WROWS
            pltpu.sync_copy(shared.at[pl.ds(rb, WROWS)],
                            out_hbm.at[pl.ds(c * N_NODES + rb, WROWS)])


def _node_body(h_ref, agg_ref, w_ref, b_ref, out_ref):
    hp0 = h_ref[0] + agg_ref[0]
    hp1 = h_ref[1] + agg_ref[1]
    acc = jnp.dot(hp0, w_ref[0:DH, :], preferred_element_type=_f32)
    acc += jnp.dot(hp1, w_ref[DH:2 * DH, :], preferred_element_type=_f32)
    out_ref[...] = jnp.maximum(acc + b_ref[...], 0.0)


def _node_update(h3, agg3, W, b2):
    nblk = N_NODES // NB
    return pl.pallas_call(
        _node_body,
        grid=(2, nblk),
        in_specs=[
            pl.BlockSpec((2, NB, DH), lambda c2, j: (0, j, 0)),
            pl.BlockSpec((2, NB, DH), lambda c2, j: (0, j, 0)),
            pl.BlockSpec((D, DH), lambda c2, j: (0, c2)),
            pl.BlockSpec((1, DH), lambda c2, j: (0, c2)),
        ],
        out_specs=pl.BlockSpec((NB, DH), lambda c2, j, _n=nblk: (c2 * _n + j, 0)),
        out_shape=jax.ShapeDtypeStruct((2 * N_NODES, DH), _f32),
    )(h3, agg3, W, b2)


def _head_body(h_ref, bat_ref, wh_ref, bh_ref, wv_ref, bv_ref,
               log_ref, val_ref, s0, s1, cnt):
    j = pl.program_id(0)
    nb = pl.num_programs(0)
    h0 = h_ref[0]
    h1 = h_ref[1]
    lg = jnp.dot(h0, wh_ref[0:DH, :], preferred_element_type=_f32)
    lg += jnp.dot(h1, wh_ref[DH:2 * DH, :], preferred_element_type=_f32)
    log_ref[...] = lg + bh_ref[...]

    onehot = (bat_ref[...] == lax.broadcasted_iota(jnp.int32, (1, NG), 1))
    onehot = onehot.astype(_f32)
    dn = (((0,), (0,)), ((), ()))
    ps0 = lax.dot_general(onehot, h0, dn, preferred_element_type=_f32)
    ps1 = lax.dot_general(onehot, h1, dn, preferred_element_type=_f32)
    pc = lax.dot_general(onehot, jnp.ones((NB, 1), _f32), dn,
                         preferred_element_type=_f32)

    @pl.when(j == 0)
    def _init():
        s0[...] = ps0
        s1[...] = ps1
        cnt[...] = pc

    @pl.when(j > 0)
    def _acc():
        s0[...] += ps0
        s1[...] += ps1
        cnt[...] += pc

    @pl.when(j == nb - 1)
    def _fin():
        v = jnp.dot(s0[...], wv_ref[0:DH, :], preferred_element_type=_f32)
        v += jnp.dot(s1[...], wv_ref[DH:2 * DH, :], preferred_element_type=_f32)
        v = v / jnp.maximum(cnt[...], 1.0) + bv_ref[...]
        val_ref[...] = jnp.tanh(v)


def _head(h3, batch2, Wh, bh2, Wv, bv2):
    return pl.pallas_call(
        _head_body,
        grid=(N_NODES // NB,),
        in_specs=[
            pl.BlockSpec((2, NB, DH), lambda j: (0, j, 0)),
            pl.BlockSpec((NB, 1), lambda j: (j, 0)),
            pl.BlockSpec((D, 1), lambda j: (0, 0)),
            pl.BlockSpec((1, 1), lambda j: (0, 0)),
            pl.BlockSpec((D, 1), lambda j: (0, 0)),
            pl.BlockSpec((1, 1), lambda j: (0, 0)),
        ],
        out_specs=[
            pl.BlockSpec((NB, 1), lambda j: (j, 0)),
            pl.BlockSpec((NG, 1), lambda j: (0, 0)),
        ],
        out_shape=[
            jax.ShapeDtypeStruct((N_NODES, 1), _f32),
            jax.ShapeDtypeStruct((NG, 1), _f32),
        ],
        scratch_shapes=[
            pltpu.VMEM((NG, DH), _f32),
            pltpu.VMEM((NG, DH), _f32),
            pltpu.VMEM((NG, 1), _f32),
        ],
    )(h3, batch2, Wh, bh2, Wv, bv2)


def kernel(x, edge_index, edge_attr, batch,
           We0, be0, W0, b0, We1, be1, W1, b1, We2, be2, W2, b2,
           Wh, bh, Wv, bv):
    src = edge_index[0].astype(jnp.int32)
    dst = edge_index[1].astype(jnp.int32)
    batch2 = batch.astype(jnp.int32).reshape(N_NODES, 1)

    h = jnp.concatenate([x[:, :DH], x[:, DH:]], axis=0)
    layers = [(We0, be0, W0, b0), (We1, be1, W1, b1), (We2, be2, W2, b2)]
    for We, be, W, b in layers:
        e_flat = _edge_mlp(edge_attr, We, be.reshape(1, D))
        agg = _sc_edge(h, e_flat, src, dst)
        h = _node_update(h.reshape(2, N_NODES, DH),
                         agg.reshape(2, N_NODES, DH), W, b.reshape(1, D))

    logits2, value2 = _head(h.reshape(2, N_NODES, DH), batch2,
                            Wh, bh.reshape(1, 1), Wv, bv.reshape(1, 1))
    return logits2.ravel(), value2.ravel()
